# baseline (speedup 1.0000x reference)
import functools

import jax
import jax.numpy as jnp
import numpy as np
from jax import lax
from jax.experimental import pallas as pl
from jax.experimental.pallas import tpu as pltpu
from jax.experimental.pallas import tpu_sc as plsc

_NNEG = 10
_TEMP = 0.1
_B, _C, _S = 8, 256, 2048
_K = 4
_POOL = _B * _S
_R = _K * _B * _S

_NCORES, _NSUB = 2, 16
_NW = _NCORES * _NSUB
_RPT = _R // _NW
_WAVE = 8
_NWAVES = _RPT // _WAVE


def _build_consts():
    rkey = jax.random.key(42)
    idx_list, w_list = [], []
    m = jnp.arange(_S)
    for k in range(1, _K + 1):
        num_pos = _B * (_S - k)
        nidx = jax.random.randint(
            jax.random.fold_in(rkey, k), (num_pos, _NNEG), 0, _POOL)
        nidx = jnp.pad(nidx.reshape(_B, _S - k, _NNEG),
                       ((0, 0), (0, k), (0, 0)))
        valid = m < _S - k
        idx_list.append(nidx)
        w_list.append(jnp.where(valid[None, :], 1.0 / (_K * num_pos), 0.0)
                      * jnp.ones((_B, 1)))
    idx = jnp.stack(idx_list).reshape(-1).astype(jnp.int32)
    w = jnp.stack(w_list).reshape(-1).astype(jnp.float32)
    return idx, w


def _pack_rows(x):
    bits = lax.bitcast_convert_type(x, jnp.uint32)
    rnd = (bits + jnp.uint32(0x7FFF) + ((bits >> 16) & jnp.uint32(1))) >> 16
    w = rnd[:128, :] | (rnd[128:, :] << 16)
    return lax.bitcast_convert_type(w, jnp.int32).T


def _prep_body(p_ref, z_ref, cp_out, z_out, pos_out):
    k = pl.program_id(0) // _B + 1
    pb = p_ref[...][0]
    zb = z_ref[...][0]
    cp_out[...] = _pack_rows(pb)[None]
    z_out[...] = _pack_rows(zb)[None]
    zsh = pltpu.roll(zb, _S - k, 1)
    pos_out[...] = (jnp.sum(zsh * pb, axis=0) * (1.0 / _TEMP))[None, None]


def _sc_body(z_hbm, cp_hbm, idx_hbm, s_hbm, m_hbm,
             idx_v, rows0, rows1, cp0, cp1, accs_m, s_v, m_v,
             sg0, sg1, sp0, sp1):
    wid = lax.axis_index("c") * _NSUB + lax.axis_index("s")
    row0 = wid * _RPT
    pltpu.sync_copy(idx_hbm.at[pl.ds(wid * (_RPT * _NNEG), _RPT * _NNEG)], idx_v)
    lane = lax.iota(jnp.int32, 16)
    lane16 = lane * 16
    bufs = ((rows0, cp0, sg0, sp0), (rows1, cp1, sg1, sp1))

    def fire(wv, rows_b, cp_b, sg, sp):
        pltpu.async_copy(
            z_hbm.at[idx_v.at[pl.ds(wv * (_WAVE * _NNEG), _WAVE * _NNEG)]],
            rows_b, sg)
        pltpu.async_copy(cp_hbm.at[pl.ds(row0 + wv * _WAVE, _WAVE)], cp_b, sp)

    def wait_bufs(rows_b, cp_b, sg, sp):
        pltpu.make_async_copy(z_hbm.at[pl.ds(0, _WAVE * _NNEG)], rows_b, sg).wait()
        pltpu.make_async_copy(cp_hbm.at[pl.ds(0, _WAVE)], cp_b, sp).wait()

    def compute(wv, rows_v, cp_v):
        s_vec = jnp.zeros((16,), jnp.float32)
        m_vec = jnp.zeros((16,), jnp.float32)
        for r in range(_WAVE):
            cpc = [plsc.bitcast(cp_v[r, pl.ds(16 * h, 16)], jnp.bfloat16)
                   for h in range(8)]
            for j in range(_NNEG):
                rj = r * _NNEG + j
                ts = [plsc.bitcast(rows_v[rj, pl.ds(16 * h, 16)],
                                   jnp.bfloat16) * cpc[h]
                      for h in range(8)]
                acc2 = ((ts[0] + ts[1]) + (ts[2] + ts[3])) + \
                       ((ts[4] + ts[5]) + (ts[6] + ts[7]))
                lo, hi = plsc.unpack(acc2,
                                     format=plsc.PackFormat.INTERLEAVED,
                                     preferred_element_type=jnp.float32)
                accs_m[pl.ds(16 * j, 16)] = lo + hi
            gs = [plsc.load_gather(accs_m, [lane16 + i]) for i in range(16)]
            for step in (8, 4, 2, 1):
                gs = [gs[i] + gs[i + step] for i in range(step)]
            lvec = gs[0] * (1.0 / _TEMP)
            lvec = jnp.where(lane < _NNEG, lvec, -1e30)
            mx = jnp.max(lvec)
            ssum = jnp.sum(jnp.exp(lvec - mx))
            s_vec = jnp.where(lane == r, ssum, s_vec)
            m_vec = jnp.where(lane == r, mx, m_vec)
        s_v[pl.ds(wv * _WAVE, 16)] = s_vec
        m_v[pl.ds(wv * _WAVE, 16)] = m_vec

    fire(0, *bufs[0])
    fire(1, *bufs[1])

    def pair(g, carry):
        for b in range(2):
            wv = 2 * g + b
            rows_b, cp_b, sg, sp = bufs[b]
            wait_bufs(rows_b, cp_b, sg, sp)
            compute(wv, rows_b, cp_b)

            @pl.when(g < _NWAVES // 2 - 1)
            def _():
                fire(wv + 2, rows_b, cp_b, sg, sp)
        return carry

    lax.fori_loop(0, _NWAVES // 2, pair, 0)
    pltpu.sync_copy(s_v.at[pl.ds(0, _RPT)], s_hbm.at[pl.ds(row0, _RPT)])
    pltpu.sync_copy(m_v.at[pl.ds(0, _RPT)], m_hbm.at[pl.ds(row0, _RPT)])


def _fin_body(s_ref, m_ref, p_ref, w_ref, o_ref):
    s, mx, p, w = s_ref[...], m_ref[...], p_ref[...], w_ref[...]
    big = jnp.maximum(mx, p)
    lse = jnp.log(jnp.exp(mx - big) * s + jnp.exp(p - big)) + big
    o_ref[...] = jnp.reshape(jnp.sum(w * (lse - p)), (1, 1))


def kernel(z, c, predictions):
    del c
    idx_arr, w_arr = _build_consts()
    preds = predictions.reshape(_K * _B, _C, _S)

    prep = pl.pallas_call(
        _prep_body,
        grid=(_K * _B,),
        in_specs=[
            pl.BlockSpec((1, _C, _S), lambda i: (i, 0, 0)),
            pl.BlockSpec((1, _C, _S), lambda i: (lax.rem(i, _B), 0, 0)),
        ],
        out_specs=[
            pl.BlockSpec((1, _S, _C // 2), lambda i: (i, 0, 0)),
            pl.BlockSpec((1, _S, _C // 2), lambda i: (lax.rem(i, _B), 0, 0)),
            pl.BlockSpec((1, 1, _S), lambda i: (i, 0, 0)),
        ],
        out_shape=[
            jax.ShapeDtypeStruct((_K * _B, _S, _C // 2), jnp.int32),
            jax.ShapeDtypeStruct((_B, _S, _C // 2), jnp.int32),
            jax.ShapeDtypeStruct((_K * _B, 1, _S), jnp.float32),
        ],
    )
    cp_bits, z_bits, pos = prep(preds, z)
    cp_bits = cp_bits.reshape(_R, _C // 2)
    z_bits = z_bits.reshape(_POOL, _C // 2)

    mesh = plsc.VectorSubcoreMesh(core_axis_name="c", subcore_axis_name="s")
    sc = functools.partial(
        pl.kernel,
        out_type=(jax.ShapeDtypeStruct((_R,), jnp.float32),
                  jax.ShapeDtypeStruct((_R,), jnp.float32)),
        mesh=mesh,
        compiler_params=pltpu.CompilerParams(needs_layout_passes=False),
        scratch_types=[
            pltpu.VMEM((_RPT * _NNEG,), jnp.int32),
            pltpu.VMEM((_WAVE * _NNEG, _C // 2), jnp.int32),
            pltpu.VMEM((_WAVE * _NNEG, _C // 2), jnp.int32),
            pltpu.VMEM((_WAVE, _C // 2), jnp.int32),
            pltpu.VMEM((_WAVE, _C // 2), jnp.int32),
            pltpu.VMEM((256,), jnp.float32),
            pltpu.VMEM((_RPT + 16,), jnp.float32),
            pltpu.VMEM((_RPT + 16,), jnp.float32),
            pltpu.SemaphoreType.DMA,
            pltpu.SemaphoreType.DMA,
            pltpu.SemaphoreType.DMA,
            pltpu.SemaphoreType.DMA,
        ],
    )(_sc_body)
    s, mx = sc(z_bits, cp_bits, idx_arr)

    fin = pl.pallas_call(
        _fin_body,
        out_shape=jax.ShapeDtypeStruct((1, 1), jnp.float32),
    )
    loss = fin(s.reshape(512, 128), mx.reshape(512, 128),
               pos.reshape(512, 128), w_arr.reshape(512, 128))
    return loss[0, 0]
